# Initial kernel scaffold; baseline (speedup 1.0000x reference)
import jax
import jax.numpy as jnp
from jax.experimental import pallas as pl


def kernel(x, timestamps, pos_embedding, time_scale):
    raise NotImplementedError("write your pallas kernel here")


# speedup vs baseline: 2.6548x; 2.6548x over previous
import functools

import jax
import jax.numpy as jnp
from jax import lax
from jax.experimental import pallas as pl
from jax.experimental.pallas import tpu as pltpu
from jax.experimental.pallas import tpu_sc as plsc

_NC = 2
_NS = 16
_NW = _NC * _NS
_L = 16

_B, _SEQ, _D = 4096, 200, 64
_TOKENS = _B * _SEQ
_VOCAB = 5000
_TOK_PER_W = _TOKENS // _NW
_BLK = 512
_NBLK = _TOK_PER_W // _BLK
_GCH = 128
_NG = _BLK // _GCH


def _prep_body(ts_ref, table_ref, scale_ref, mm_ref, stable_ref):
    t = ts_ref[...]
    tmin = jnp.min(t)
    trange = jnp.max(t) - tmin
    safe = jnp.where(trange > 0, trange, jnp.float32(1.0))
    row = lax.broadcasted_iota(jnp.int32, (8, 128), 0)
    mm_ref[...] = jnp.where(row == 0, tmin, safe)
    stable_ref[...] = table_ref[...] * scale_ref[...]


_prep = pl.pallas_call(
    _prep_body,
    out_shape=[
        jax.ShapeDtypeStruct((8, 128), jnp.float32),
        jax.ShapeDtypeStruct((_VOCAB, _D), jnp.float32),
    ],
)


def _sc_body(x_hbm, ts_hbm, stable_hbm, mm_hbm, out_hbm, mm_v, ts_v, idx_v, x_v, sem):
    wid = lax.axis_index("s") * _NC + lax.axis_index("c")
    base0 = wid * _TOK_PER_W

    pltpu.sync_copy(mm_hbm.at[pl.ds(0, _L)], mm_v.at[pl.ds(0, _L)])
    pltpu.sync_copy(mm_hbm.at[pl.ds(128, _L)], mm_v.at[pl.ds(_L, _L)])
    tmin = mm_v[pl.ds(0, _L)]
    tsafe = mm_v[pl.ds(_L, _L)]

    def block(b, carry):
        tok = pl.multiple_of(base0 + b * _BLK, _BLK)
        pltpu.sync_copy(ts_hbm.at[pl.ds(tok, _BLK)], ts_v)
        for k in range(_BLK // _L):
            t = ts_v[pl.ds(k * _L, _L)]
            p = (t - tmin) / tsafe * jnp.float32(4999.0)
            g, o = divmod(k * _L, _GCH)
            idx_v[g, pl.ds(o, _L)] = p.astype(jnp.int32)
        pltpu.sync_copy(x_hbm.at[pl.ds(tok, _BLK)], x_v)
        copies = [
            pltpu.async_copy(
                stable_hbm.at[idx_v.at[g]],
                x_v.at[pl.ds(g * _GCH, _GCH)],
                sem,
                add=True,
            )
            for g in range(_NG)
        ]
        for c in copies:
            c.wait()
        pltpu.sync_copy(x_v, out_hbm.at[pl.ds(tok, _BLK)])
        return carry

    lax.fori_loop(0, _NBLK, block, 0)


_sc = functools.partial(
    pl.kernel,
    out_type=jax.ShapeDtypeStruct((_TOKENS, _D), jnp.float32),
    mesh=plsc.VectorSubcoreMesh(core_axis_name="c", subcore_axis_name="s"),
    scratch_types=[
        pltpu.VMEM((2 * _L,), jnp.float32),
        pltpu.VMEM((_BLK,), jnp.float32),
        pltpu.VMEM((_NG, _GCH), jnp.int32),
        pltpu.VMEM((_BLK, _D), jnp.float32),
        pltpu.SemaphoreType.DMA,
    ],
    compiler_params=pltpu.CompilerParams(use_tc_tiling_on_sc=False),
)(_sc_body)


def kernel(x, timestamps, pos_embedding, time_scale):
    mm, stable = _prep(
        timestamps, pos_embedding, time_scale.reshape(1, 1).astype(jnp.float32)
    )
    out = _sc(
        x.reshape(_TOKENS, _D),
        timestamps.reshape(_TOKENS),
        stable,
        mm.reshape(8 * 128),
    )
    return out.reshape(x.shape)


# speedup vs baseline: 2.8536x; 1.0749x over previous
import functools

import jax
import jax.numpy as jnp
from jax import lax
from jax.experimental import pallas as pl
from jax.experimental.pallas import tpu as pltpu
from jax.experimental.pallas import tpu_sc as plsc

_NC = 2
_NS = 16
_NW = _NC * _NS
_L = 16

_B, _SEQ, _D = 4096, 200, 64
_TOKENS = _B * _SEQ
_VOCAB = 5000
_TOK_PER_W = _TOKENS // _NW
_BLK = 512
_NBLK = _TOK_PER_W // _BLK
_GCH = 128
_NG = _BLK // _GCH


def _prep_body(ts_ref, table_ref, scale_ref, mm_ref, stable_ref):
    t = ts_ref[...]
    tmin = jnp.min(t)
    trange = jnp.max(t) - tmin
    safe = jnp.where(trange > 0, trange, jnp.float32(1.0))
    row = lax.broadcasted_iota(jnp.int32, (8, 128), 0)
    mm_ref[...] = jnp.where(row == 0, tmin, safe)
    stable_ref[...] = table_ref[...] * scale_ref[...]


_prep = pl.pallas_call(
    _prep_body,
    out_shape=[
        jax.ShapeDtypeStruct((8, 128), jnp.float32),
        jax.ShapeDtypeStruct((_VOCAB, _D), jnp.float32),
    ],
)


def _sc_body(
    x_hbm, ts_hbm, stable_hbm, mm_hbm, out_hbm,
    mm_v, ts_v, idx_v, x_v,
    sem_ts0, sem_x0, sem_g0, sem_o0, sem_ts1, sem_x1, sem_g1, sem_o1,
):
    wid = lax.axis_index("s") * _NC + lax.axis_index("c")
    base0 = wid * _TOK_PER_W
    sem_ts = (sem_ts0, sem_ts1)
    sem_x = (sem_x0, sem_x1)
    sem_g = (sem_g0, sem_g1)
    sem_o = (sem_o0, sem_o1)

    pltpu.sync_copy(mm_hbm.at[pl.ds(0, _L)], mm_v.at[pl.ds(0, _L)])
    pltpu.sync_copy(mm_hbm.at[pl.ds(128, _L)], mm_v.at[pl.ds(_L, _L)])
    tmin = mm_v[pl.ds(0, _L)]
    tsafe = mm_v[pl.ds(_L, _L)]

    def tok_of(b):
        return pl.multiple_of(base0 + b * _BLK, _BLK)

    def start_loads(p, b):
        tok = tok_of(b)
        pltpu.async_copy(ts_hbm.at[pl.ds(tok, _BLK)], ts_v.at[p], sem_ts[p])
        pltpu.async_copy(x_hbm.at[pl.ds(tok, _BLK)], x_v.at[p], sem_x[p])

    def wait_writeback(p):
        pltpu.make_async_copy(x_v.at[p], out_hbm.at[pl.ds(0, _BLK)], sem_o[p]).wait()

    def compute_idx(p):
        pltpu.make_async_copy(ts_hbm.at[pl.ds(0, _BLK)], ts_v.at[p], sem_ts[p]).wait()
        for k in range(_BLK // _L):
            t = ts_v[p, pl.ds(k * _L, _L)]
            v = (t - tmin) / tsafe * jnp.float32(4999.0)
            g, o = divmod(k * _L, _GCH)
            idx_v[p, g, pl.ds(o, _L)] = v.astype(jnp.int32)

    def fire_gathers(p):
        pltpu.make_async_copy(x_hbm.at[pl.ds(0, _BLK)], x_v.at[p], sem_x[p]).wait()
        return [
            pltpu.async_copy(
                stable_hbm.at[idx_v.at[p, g]],
                x_v.at[p, pl.ds(g * _GCH, _GCH)],
                sem_g[p],
                add=True,
            )
            for g in range(_NG)
        ]

    def start_writeback(p, b, gathers):
        for c in gathers:
            c.wait()
        pltpu.async_copy(x_v.at[p], out_hbm.at[pl.ds(tok_of(b), _BLK)], sem_o[p])

    def body(i, carry):
        b0, b1 = 2 * i, 2 * i + 1

        @pl.when(i > 0)
        def _():
            wait_writeback(0)

        start_loads(0, b0)

        @pl.when(i > 0)
        def _():
            wait_writeback(1)

        start_loads(1, b1)
        compute_idx(0)
        g0 = fire_gathers(0)
        compute_idx(1)
        start_writeback(0, b0, g0)
        g1 = fire_gathers(1)
        start_writeback(1, b1, g1)
        return carry

    lax.fori_loop(0, _NBLK // 2, body, 0)
    wait_writeback(0)
    wait_writeback(1)


_sc = functools.partial(
    pl.kernel,
    out_type=jax.ShapeDtypeStruct((_TOKENS, _D), jnp.float32),
    mesh=plsc.VectorSubcoreMesh(core_axis_name="c", subcore_axis_name="s"),
    scratch_types=[
        pltpu.VMEM((2 * _L,), jnp.float32),
        pltpu.VMEM((2, _BLK), jnp.float32),
        pltpu.VMEM((2, _NG, _GCH), jnp.int32),
        pltpu.VMEM((2, _BLK, _D), jnp.float32),
    ] + [pltpu.SemaphoreType.DMA] * 8,
    compiler_params=pltpu.CompilerParams(use_tc_tiling_on_sc=False),
)(_sc_body)


def kernel(x, timestamps, pos_embedding, time_scale):
    mm, stable = _prep(
        timestamps, pos_embedding, time_scale.reshape(1, 1).astype(jnp.float32)
    )
    out = _sc(
        x.reshape(_TOKENS, _D),
        timestamps.reshape(_TOKENS),
        stable,
        mm.reshape(8 * 128),
    )
    return out.reshape(x.shape)
